# baseline (speedup 1.0000x reference)
import functools

import jax
import jax.numpy as jnp
from jax import lax
from jax.experimental import pallas as pl
from jax.experimental.pallas import tpu as pltpu
from jax.experimental.pallas import tpu_sc as plsc

VOCAB_SIZE = 49408
HIDDEN_SIZE = 512
MAX_POS = 77
BATCH = 4096
SEQ = 77

N = BATCH * SEQ
NC = 2
NS = 16
NW = NC * NS
PER_W = N // NW
C = 64
NCHUNK = PER_W // C
NBUF = 2
LANES = 16
NGROUP = C // LANES
CUNROLL = 4

assert PER_W * NW == N and NCHUNK * C == PER_W and NCHUNK % NBUF == 0
assert HIDDEN_SIZE % CUNROLL == 0

_mesh = plsc.VectorSubcoreMesh(core_axis_name="c", subcore_axis_name="s")


@functools.partial(
    pl.kernel,
    out_type=jax.ShapeDtypeStruct((N, HIDDEN_SIZE), jnp.float32),
    mesh=_mesh,
    compiler_params=pltpu.CompilerParams(use_tc_tiling_on_sc=False, needs_layout_passes=False),
    scratch_types=[
        pltpu.VMEM((PER_W,), jnp.int32),
        pltpu.VMEM((PER_W,), jnp.int32),
        pltpu.VMEM((MAX_POS, HIDDEN_SIZE), jnp.float32),
        pltpu.VMEM((C, HIDDEN_SIZE), jnp.float32),
        pltpu.VMEM((C, HIDDEN_SIZE), jnp.float32),
        pltpu.SemaphoreType.DMA,
        pltpu.SemaphoreType.DMA,
        pltpu.SemaphoreType.DMA,
        pltpu.SemaphoreType.DMA,
    ],
)
def _emb_lookup(ids_hbm, pids_hbm, tok_hbm, pos_hbm, out_hbm,
                idx_t, idx_p, pos_v, buf0, buf1,
                semt0, semt1, semo0, semo1):
    wid = lax.axis_index("s") * NC + lax.axis_index("c")
    w_base = wid * PER_W

    bufs = (buf0, buf1)
    semt = (semt0, semt1)
    semo = (semo0, semo1)

    pltpu.sync_copy(ids_hbm.at[pl.ds(w_base, PER_W)], idx_t)
    pltpu.sync_copy(pids_hbm.at[pl.ds(w_base, PER_W)], idx_p)
    pltpu.sync_copy(pos_hbm, pos_v)

    def tok_copy(g, b):
        return pltpu.make_async_copy(
            tok_hbm.at[idx_t.at[pl.ds(g * C, C)]], bufs[b], semt[b])

    def out_copy(g, b):
        return pltpu.make_async_copy(
            bufs[b], out_hbm.at[pl.ds(w_base + g * C, C)], semo[b])

    row_idx = [lax.iota(jnp.int32, LANES) + (k * LANES) for k in range(NGROUP)]

    tok_copy(0, 0).start()

    def superstep(kk, carry):
        for b in range(NBUF):
            g = NBUF * kk + b
            ob = 1 - b
            tok_copy(g, b).wait()

            @pl.when(g >= 1)
            def _():
                out_copy(g - 1, ob).wait()

            @pl.when(g + 1 < NCHUNK)
            def _():
                tok_copy(g + 1, ob).start()

            p_vecs = [idx_p[pl.ds(g * C + k * LANES, LANES)]
                      for k in range(NGROUP)]

            def add_cols(ci, carry2):
                for u in range(CUNROLL):
                    col = ci * CUNROLL + u
                    c_splat = jnp.full((LANES,), 0, jnp.int32) + col
                    for k in range(NGROUP):
                        v = plsc.load_gather(pos_v, [p_vecs[k], c_splat])
                        plsc.addupdate_scatter(bufs[b], [row_idx[k], c_splat], v)
                return carry2

            lax.fori_loop(0, HIDDEN_SIZE // CUNROLL, add_cols, 0)
            out_copy(g, b).start()
        return carry

    lax.fori_loop(0, NCHUNK // NBUF, superstep, 0)
    out_copy(NCHUNK - 1, (NCHUNK - 1) % NBUF).wait()


def kernel(input_ids, position_ids, token_embedding, position_embedding):
    ids = input_ids.reshape(N).astype(jnp.int32)
    pids = position_ids.reshape(N).astype(jnp.int32)
    out = _emb_lookup(ids, pids, token_embedding, position_embedding)
    return out.reshape(BATCH, SEQ, HIDDEN_SIZE)


# speedup vs baseline: 3.2331x; 3.2331x over previous
import functools

import jax
import jax.numpy as jnp
from jax import lax
from jax.experimental import pallas as pl
from jax.experimental.pallas import tpu as pltpu
from jax.experimental.pallas import tpu_sc as plsc

VOCAB_SIZE = 49408
HIDDEN_SIZE = 512
MAX_POS = 77
BATCH = 4096
SEQ = 77

N = BATCH * SEQ
NC = 2
NS = 16
NW = NC * NS
PER_W = N // NW
C = 64
NCHUNK = PER_W // C
NBUF = 2
LANES = 16
NGROUP = C // LANES
CUNROLL = 4

assert PER_W * NW == N and NCHUNK * C == PER_W and NCHUNK % NBUF == 0
assert HIDDEN_SIZE % CUNROLL == 0

_mesh = plsc.VectorSubcoreMesh(core_axis_name="c", subcore_axis_name="s")


@functools.partial(
    pl.kernel,
    out_type=jax.ShapeDtypeStruct((N, HIDDEN_SIZE), jnp.float32),
    mesh=_mesh,
    compiler_params=pltpu.CompilerParams(use_tc_tiling_on_sc=False, needs_layout_passes=False),
    scratch_types=[
        pltpu.VMEM((PER_W,), jnp.int32),
        pltpu.VMEM((PER_W,), jnp.int32),
        pltpu.VMEM((MAX_POS, HIDDEN_SIZE), jnp.float32),
        pltpu.VMEM((C, HIDDEN_SIZE), jnp.float32),
        pltpu.VMEM((C, HIDDEN_SIZE), jnp.float32),
        pltpu.SemaphoreType.DMA,
        pltpu.SemaphoreType.DMA,
        pltpu.SemaphoreType.DMA,
        pltpu.SemaphoreType.DMA,
    ],
)
def _emb_lookup(ids_hbm, pids_hbm, tok_hbm, pos_hbm, out_hbm,
                idx_t, idx_p, pos_v, buf0, buf1,
                semt0, semt1, semo0, semo1):
    wid = lax.axis_index("s") * NC + lax.axis_index("c")
    w_base = wid * PER_W

    bufs = (buf0, buf1)
    semt = (semt0, semt1)
    semo = (semo0, semo1)

    pltpu.sync_copy(ids_hbm.at[pl.ds(w_base, PER_W)], idx_t)
    pltpu.sync_copy(pids_hbm.at[pl.ds(w_base, PER_W)], idx_p)
    pltpu.sync_copy(pos_hbm, pos_v)

    def tok_copy(g, b):
        return pltpu.make_async_copy(
            tok_hbm.at[idx_t.at[pl.ds(g * C, C)]], bufs[b], semt[b])

    def out_copy(g, b):
        return pltpu.make_async_copy(
            bufs[b], out_hbm.at[pl.ds(w_base + g * C, C)], semo[b])

    tok_copy(0, 0).start()

    def superstep(kk, carry):
        for b in range(NBUF):
            g = NBUF * kk + b
            ob = 1 - b
            tok_copy(g, b).wait()

            @pl.when(g >= 1)
            def _():
                out_copy(g - 1, ob).wait()

            @pl.when(g + 1 < NCHUNK)
            def _():
                tok_copy(g + 1, ob).start()

            def add_group(k, carry2):
                p_vec = idx_p[pl.ds(g * C + k * LANES, LANES)]
                for r16 in range(LANES):
                    p_r = p_vec[r16]
                    r = k * LANES + r16
                    for j in range(HIDDEN_SIZE // LANES):
                        sl = pl.ds(j * LANES, LANES)
                        plsc.addupdate(bufs[b].at[r, sl], pos_v[p_r, sl])
                return carry2

            lax.fori_loop(0, NGROUP, add_group, 0)
            out_copy(g, b).start()
        return carry

    lax.fori_loop(0, NCHUNK // NBUF, superstep, 0)
    out_copy(NCHUNK - 1, (NCHUNK - 1) % NBUF).wait()


def kernel(input_ids, position_ids, token_embedding, position_embedding):
    ids = input_ids.reshape(N).astype(jnp.int32)
    pids = position_ids.reshape(N).astype(jnp.int32)
    out = _emb_lookup(ids, pids, token_embedding, position_embedding)
    return out.reshape(BATCH, SEQ, HIDDEN_SIZE)
